# speedup vs baseline: 10.5095x; 10.5095x over previous
import functools

import jax
import jax.numpy as jnp
from jax.experimental import pallas as pl


def _body(day_ref, week_ref, table_ref, out_ref, *, n_day, k_pad, nb):
    day = day_ref[0]
    week = week_ref[0]
    idx_d = (day * float(n_day)).astype(jnp.int32)
    idx_w = week.astype(jnp.int32) + n_day
    iota = jax.lax.broadcasted_iota(jnp.int32, (k_pad, nb), 0)
    two_hot = (iota == idx_d).astype(jnp.float32) + (iota == idx_w).astype(
        jnp.float32
    )
    out_ref[0] = jnp.dot(
        table_ref[...], two_hot, preferred_element_type=jnp.float32
    )


def kernel(x, time_day, time_week, time):
    B, T, N, C = x.shape
    n_day, F = time_day.shape
    n_week = time_week.shape[0]
    k = n_day + n_week
    k_pad = (k + 7) // 8 * 8

    table = jnp.concatenate(
        [time_day, time_week, jnp.zeros((k_pad - k, F), jnp.float32)], axis=0
    )
    table_t = table.T

    day = x[:, -1, :, 1][:, None, :]
    week = x[:, -1, :, 2][:, None, :]

    nb = N
    grid = (B, N // nb)

    out = pl.pallas_call(
        functools.partial(_body, n_day=n_day, k_pad=k_pad, nb=nb),
        grid=grid,
        in_specs=[
            pl.BlockSpec((1, 1, nb), lambda b, n: (b, 0, n)),
            pl.BlockSpec((1, 1, nb), lambda b, n: (b, 0, n)),
            pl.BlockSpec((F, k_pad), lambda b, n: (0, 0)),
        ],
        out_specs=pl.BlockSpec((1, F, nb), lambda b, n: (b, 0, n)),
        out_shape=jax.ShapeDtypeStruct((B, F, N), jnp.float32),
    )(day, week, table_t)

    return out[..., None]


# speedup vs baseline: 11.3470x; 1.0797x over previous
import functools

import jax
import jax.numpy as jnp
from jax.experimental import pallas as pl


def _body(idx_d_ref, idx_w_ref, table_ref, out_ref, *, n_day, k_pad, nb):
    idx_d = idx_d_ref[0]
    idx_w = idx_w_ref[0]
    iota_d = jax.lax.broadcasted_iota(jnp.int32, (n_day, nb), 0)
    iota_w = jax.lax.broadcasted_iota(jnp.int32, (k_pad - n_day, nb), 0)
    oh_d = (iota_d == idx_d).astype(jnp.bfloat16)
    oh_w = (iota_w == idx_w).astype(jnp.bfloat16)
    one_hot = jnp.concatenate([oh_d, oh_w], axis=0)
    out_ref[0] = jnp.dot(
        table_ref[...], one_hot, preferred_element_type=jnp.float32
    )


def kernel(x, time_day, time_week, time):
    B, T, N, C = x.shape
    n_day, F = time_day.shape
    n_week = time_week.shape[0]
    k_pad = n_day + (n_week + 7) // 8 * 8

    table = jnp.concatenate(
        [time_day, time_week, jnp.zeros((k_pad - n_day - n_week, F), jnp.float32)],
        axis=0,
    )
    table_t = table.T.astype(jnp.bfloat16)

    idx_d = (x[:, -1, :, 1] * time).astype(jnp.int32)[:, None, :]
    idx_w = x[:, -1, :, 2].astype(jnp.int32)[:, None, :]

    nb = N
    grid = (B,)

    out = pl.pallas_call(
        functools.partial(_body, n_day=n_day, k_pad=k_pad, nb=nb),
        grid=grid,
        in_specs=[
            pl.BlockSpec((1, 1, nb), lambda b: (b, 0, 0)),
            pl.BlockSpec((1, 1, nb), lambda b: (b, 0, 0)),
            pl.BlockSpec((F, k_pad), lambda b: (0, 0)),
        ],
        out_specs=pl.BlockSpec((1, F, nb), lambda b: (b, 0, 0)),
        out_shape=jax.ShapeDtypeStruct((B, F, N), jnp.float32),
    )(idx_d, idx_w, table_t)

    return out[..., None]


# speedup vs baseline: 29.2475x; 2.5775x over previous
import functools

import jax
import jax.numpy as jnp
from jax.experimental import pallas as pl
from jax.experimental.pallas import tpu as pltpu

_SLOTS = 4


def _body(
    idx_d_ref,
    idx_w_ref,
    table_ref,
    out_ref,
    scratch,
    sems,
    *,
    n_day,
    k_pad,
    nb,
    bb,
    num_steps,
):
    f = table_ref.shape[0]
    fh = f // 2
    s = pl.program_id(0)
    iota_d = jax.lax.broadcasted_iota(jnp.int32, (n_day, nb), 0)
    iota_w = jax.lax.broadcasted_iota(jnp.int32, (k_pad - n_day, nb), 0)

    def _waits(slot, b_prev):
        for h in range(2):
            pltpu.make_async_copy(
                scratch.at[slot, h * fh : (h + 1) * fh],
                out_ref.at[b_prev, h * fh : (h + 1) * fh],
                sems.at[slot, h],
            ).wait()

    for j in range(bb):
        slot = j % _SLOTS
        b = s * bb + j
        if j >= _SLOTS:
            _waits(slot, b - _SLOTS)
        else:

            @pl.when(s > 0)
            def _():
                _waits(slot, b - _SLOTS)

        idx_d = idx_d_ref[0, pl.ds(j, 1), :]
        idx_w = idx_w_ref[0, pl.ds(j, 1), :]
        oh_d = (iota_d == idx_d).astype(jnp.bfloat16)
        oh_w = (iota_w == idx_w).astype(jnp.bfloat16)
        one_hot = jnp.concatenate([oh_d, oh_w], axis=0)
        res = jnp.dot(
            table_ref[...], one_hot, preferred_element_type=jnp.float32
        )
        scratch[slot] = res.reshape(f, nb // 128, 128)
        for h in range(2):
            pltpu.make_async_copy(
                scratch.at[slot, h * fh : (h + 1) * fh],
                out_ref.at[b, h * fh : (h + 1) * fh],
                sems.at[slot, h],
            ).start()

    @pl.when(s == num_steps - 1)
    def _():
        for j in range(bb - _SLOTS, bb):
            _waits(j % _SLOTS, s * bb + j)


def kernel(x, time_day, time_week, time):
    B, T, N, C = x.shape
    n_day, F = time_day.shape
    n_week = time_week.shape[0]
    k_pad = n_day + (n_week + 7) // 8 * 8

    table = jnp.concatenate(
        [time_day, time_week, jnp.zeros((k_pad - n_day - n_week, F), jnp.float32)],
        axis=0,
    )
    table_t = table.T.astype(jnp.bfloat16)

    bb = 8
    nb = N
    num_steps = B // bb
    grid = (num_steps,)

    idx_d = (x[:, -1, :, 1] * time).astype(jnp.int32).reshape(B // bb, bb, N)
    idx_w = x[:, -1, :, 2].astype(jnp.int32).reshape(B // bb, bb, N)

    out = pl.pallas_call(
        functools.partial(
            _body, n_day=n_day, k_pad=k_pad, nb=nb, bb=bb, num_steps=num_steps
        ),
        grid=grid,
        in_specs=[
            pl.BlockSpec((1, bb, nb), lambda b: (b, 0, 0)),
            pl.BlockSpec((1, bb, nb), lambda b: (b, 0, 0)),
            pl.BlockSpec((F, k_pad), lambda b: (0, 0)),
        ],
        out_specs=pl.BlockSpec(memory_space=pltpu.MemorySpace.HBM),
        out_shape=jax.ShapeDtypeStruct((B, F, N // 128, 128), jnp.float32),
        scratch_shapes=[
            pltpu.VMEM((_SLOTS, F, N // 128, 128), jnp.float32),
            pltpu.SemaphoreType.DMA((_SLOTS, 2)),
        ],
    )(idx_d, idx_w, table_t)

    return out.reshape(B, F, N, 1)


# speedup vs baseline: 30.5323x; 1.0439x over previous
import functools

import jax
import jax.numpy as jnp
from jax.experimental import pallas as pl
from jax.experimental.pallas import tpu as pltpu


def _body(idx_d_ref, idx_w_ref, table_ref, out_ref, *, n_day, k_pad, nb, bb):
    f = table_ref.shape[0]
    iota_d = jax.lax.broadcasted_iota(jnp.int32, (n_day, nb), 0)
    iota_w = jax.lax.broadcasted_iota(jnp.int32, (k_pad - n_day, nb), 0)
    for j in range(bb):
        idx_d = idx_d_ref[0, pl.ds(j, 1), :]
        idx_w = idx_w_ref[0, pl.ds(j, 1), :]
        oh_d = (iota_d == idx_d).astype(jnp.bfloat16)
        oh_w = (iota_w == idx_w).astype(jnp.bfloat16)
        one_hot = jnp.concatenate([oh_d, oh_w], axis=0)
        res = jnp.dot(
            table_ref[...], one_hot, preferred_element_type=jnp.float32
        )
        out_ref[j] = res.reshape(f, nb // 128, 128)


def kernel(x, time_day, time_week, time):
    B, T, N, C = x.shape
    n_day, F = time_day.shape
    n_week = time_week.shape[0]
    k_pad = n_day + (n_week + 7) // 8 * 8

    table = jnp.concatenate(
        [time_day, time_week, jnp.zeros((k_pad - n_day - n_week, F), jnp.float32)],
        axis=0,
    )
    table_t = table.T.astype(jnp.bfloat16)

    bb = 8
    nb = N
    grid = (B // bb,)

    idx_d = (x[:, -1, :, 1] * time).astype(jnp.int32).reshape(B // bb, bb, N)
    idx_w = x[:, -1, :, 2].astype(jnp.int32).reshape(B // bb, bb, N)

    out = pl.pallas_call(
        functools.partial(_body, n_day=n_day, k_pad=k_pad, nb=nb, bb=bb),
        grid=grid,
        in_specs=[
            pl.BlockSpec((1, bb, nb), lambda b: (b, 0, 0)),
            pl.BlockSpec((1, bb, nb), lambda b: (b, 0, 0)),
            pl.BlockSpec((F, k_pad), lambda b: (0, 0)),
        ],
        out_specs=pl.BlockSpec(
            (bb, F, nb // 128, 128), lambda b: (b, 0, 0, 0)
        ),
        out_shape=jax.ShapeDtypeStruct((B, F, N // 128, 128), jnp.float32),
        compiler_params=pltpu.CompilerParams(
            dimension_semantics=("parallel",),
        ),
    )(idx_d, idx_w, table_t)

    return out.reshape(B, F, N, 1)


# speedup vs baseline: 30.5605x; 1.0009x over previous
import functools

import jax
import jax.numpy as jnp
from jax.experimental import pallas as pl
from jax.experimental.pallas import tpu as pltpu


def _body(idx_d_ref, idx_w_ref, table_ref, out_ref, *, n_day, k_pad, nb, bb):
    f = table_ref.shape[0]
    iota_d = jax.lax.broadcasted_iota(jnp.int32, (n_day, nb), 0)
    iota_w = jax.lax.broadcasted_iota(jnp.int32, (k_pad - n_day, nb), 0)
    for j in range(bb):
        idx_d = idx_d_ref[0, pl.ds(j, 1), :]
        idx_w = idx_w_ref[0, pl.ds(j, 1), :]
        oh_d = (iota_d == idx_d).astype(jnp.bfloat16)
        oh_w = (iota_w == idx_w).astype(jnp.bfloat16)
        one_hot = jnp.concatenate([oh_d, oh_w], axis=0)
        res = jnp.dot(
            table_ref[...], one_hot, preferred_element_type=jnp.float32
        )
        out_ref[j] = res.reshape(f, nb // 128, 128)


def kernel(x, time_day, time_week, time):
    B, T, N, C = x.shape
    n_day, F = time_day.shape
    n_week = time_week.shape[0]
    k_pad = n_day + (n_week + 7) // 8 * 8

    table = jnp.concatenate(
        [time_day, time_week, jnp.zeros((k_pad - n_day - n_week, F), jnp.float32)],
        axis=0,
    )
    table_t = table.T.astype(jnp.bfloat16)

    bb = 8
    nb = N
    grid = (B // bb,)

    idx_d = (x[:, -1, :, 1] * time).astype(jnp.int32).reshape(B // bb, bb, N)
    idx_w = x[:, -1, :, 2].astype(jnp.int32).reshape(B // bb, bb, N)

    out = pl.pallas_call(
        functools.partial(_body, n_day=n_day, k_pad=k_pad, nb=nb, bb=bb),
        grid=grid,
        in_specs=[
            pl.BlockSpec((1, bb, nb), lambda b: (b, 0, 0)),
            pl.BlockSpec((1, bb, nb), lambda b: (b, 0, 0)),
            pl.BlockSpec((F, k_pad), lambda b: (0, 0)),
        ],
        out_specs=pl.BlockSpec(
            (bb, F, nb // 128, 128), lambda b: (b, 0, 0, 0)
        ),
        out_shape=jax.ShapeDtypeStruct((B, F, N // 128, 128), jnp.float32),
        compiler_params=pltpu.CompilerParams(
            dimension_semantics=("arbitrary",),
        ),
    )(idx_d, idx_w, table_t)

    return out.reshape(B, F, N, 1)
